# Initial kernel scaffold; baseline (speedup 1.0000x reference)
import jax
import jax.numpy as jnp
from jax.experimental import pallas as pl


def kernel(edge_index, features, sim, label_smoothing, W1, b1, W2, b2, Wy, by, neg_edge_index):
    raise NotImplementedError("write your pallas kernel here")


# speedup vs baseline: 2.4574x; 2.4574x over previous
import functools

import jax
import jax.numpy as jnp
from jax import lax
from jax.experimental import pallas as pl
from jax.experimental.pallas import tpu as pltpu
from jax.experimental.pallas import tpu_sc as plsc

N = 10000
F = 128
NCLS = 40
THETA = 0.5

NW = 32
CH = 128
NCH = 80
EPT = NCH * CH
EPAD = NW * EPT
ACC_ROWS = 10240
NP = ACC_ROWS
RPT = ACC_ROWS // 16

BLK = 1280
GRID = NP // BLK


def _mesh():
    return plsc.VectorSubcoreMesh(core_axis_name="c", subcore_axis_name="s")


def _deg_body(col_hbm, out_hbm, cidx, hist):
    c = lax.axis_index("c")
    s = lax.axis_index("s")
    wid = s * 2 + c
    pltpu.sync_copy(col_hbm.at[wid], cidx)
    zero16 = jnp.zeros((16,), jnp.float32)
    ones16 = jnp.ones((16,), jnp.float32)

    def zbody(i, carry):
        hist[pl.ds(i * 16, 16)] = zero16
        return carry

    lax.fori_loop(0, ACC_ROWS // 16, zbody, 0)

    def cbody(ci, carry):
        for g in range(8):
            v = cidx[ci, pl.ds(g * 16, 16)]
            plsc.addupdate_scatter(hist, [v], ones16)
        return carry

    lax.fori_loop(0, NCH, cbody, 0)
    pltpu.sync_copy(hist, out_hbm.at[wid])


def _deg(col_p):
    k = pl.kernel(
        _deg_body,
        out_type=jax.ShapeDtypeStruct((NW, ACC_ROWS), jnp.float32),
        mesh=_mesh(),
        scratch_types=[
            pltpu.VMEM((NCH, CH), jnp.int32),
            pltpu.VMEM((ACC_ROWS,), jnp.float32),
        ],
        compiler_params=pltpu.CompilerParams(needs_layout_passes=False),
    )
    return k(col_p)


IBLK = 8


def _spmm_body(xs_hbm, row_hbm, col_hbm, out_hbm,
               ridx, cidx, buf0, buf1, acc, g0, g1, s0, s1):
    c = lax.axis_index("c")
    s = lax.axis_index("s")
    wid = s * 2 + c

    zero16 = jnp.zeros((16,), jnp.float32)

    def zbody(r, carry):
        for g in range(8):
            buf0[r, pl.ds(g * 16, 16)] = zero16
        return carry

    lax.fori_loop(0, CH, zbody, 0)
    for i in range(RPT // CH):
        pltpu.sync_copy(buf0, acc.at[pl.ds(s * RPT + i * CH, CH)])
    plsc.subcore_barrier()

    bufs = (buf0, buf1)
    gsem = (g0, g1)
    ssem = (s0, s1)
    for blk in range(NCH // IBLK):
        pltpu.sync_copy(row_hbm.at[wid, pl.ds(blk * IBLK, IBLK)], ridx)
        pltpu.sync_copy(col_hbm.at[wid, pl.ds(blk * IBLK, IBLK)], cidx)
        handles = {}
        handles[("g", 0)] = pltpu.async_copy(
            xs_hbm.at[ridx.at[0]], bufs[0], gsem[0])
        for j in range(IBLK):
            b = j & 1
            handles[("g", j)].wait()
            handles[("s", j)] = pltpu.async_copy(
                bufs[b], acc.at[cidx.at[j]], ssem[b], add=True)
            if j + 1 < IBLK:
                if j >= 1:
                    handles[("s", j - 1)].wait()
                handles[("g", j + 1)] = pltpu.async_copy(
                    xs_hbm.at[ridx.at[j + 1]], bufs[1 - b], gsem[1 - b])
        handles[("s", IBLK - 2)].wait()
        handles[("s", IBLK - 1)].wait()
    plsc.subcore_barrier()

    for i in range(RPT // CH):
        rs = s * RPT + i * CH
        pltpu.sync_copy(acc.at[pl.ds(rs, CH)], buf0)
        pltpu.sync_copy(buf0, out_hbm.at[c, pl.ds(rs, CH)])


def _spmm(xs, row_p, col_p):
    k = pl.kernel(
        _spmm_body,
        out_type=jax.ShapeDtypeStruct((2, NP, F), jnp.float32),
        mesh=_mesh(),
        scratch_types=[
            pltpu.VMEM((IBLK, CH), jnp.int32),
            pltpu.VMEM((IBLK, CH), jnp.int32),
            pltpu.VMEM((CH, F), jnp.float32),
            pltpu.VMEM((CH, F), jnp.float32),
            pltpu.VMEM_SHARED((ACC_ROWS, F), jnp.float32),
            pltpu.SemaphoreType.DMA,
            pltpu.SemaphoreType.DMA,
            pltpu.SemaphoreType.DMA,
            pltpu.SemaphoreType.DMA,
        ],
    )
    return k(xs, row_p, col_p)


def _score_body(rep_hbm, sim_hbm, ps_hbm, pd_hbm, ns_hbm, nd_hbm, out_hbm,
                ps, pd, ns_, nd, arows, brows, fidx, simv, outbuf,
                sa, sb, sv):
    c = lax.axis_index("c")
    s = lax.axis_index("s")
    wid = s * 2 + c
    pltpu.sync_copy(ps_hbm.at[wid], ps)
    pltpu.sync_copy(pd_hbm.at[wid], pd)
    pltpu.sync_copy(ns_hbm.at[wid], ns_)
    pltpu.sync_copy(nd_hbm.at[wid], nd)

    zero16 = jnp.zeros((16,), jnp.float32)
    iota16 = lax.iota(jnp.int32, 16)

    def dot_group(g, av, bv):
        rows_g = g * 16 + iota16

        def dbody(d, acc):
            dvec = jnp.full((16,), 0, jnp.int32) + d
            va = plsc.load_gather(av, [rows_g, dvec])
            vb = plsc.load_gather(bv, [rows_g, dvec])
            return acc + va * vb

        return lax.fori_loop(0, F, dbody, zero16)

    def pos_chunk(ci, carry):
        pp2, pp1, pc = carry
        for g in range(8):
            sv_ = ps[ci, pl.ds(g * 16, 16)]
            dv_ = pd[ci, pl.ds(g * 16, 16)]
            fidx[0, pl.ds(g * 16, 16)] = sv_ * N + dv_
        ha = pltpu.async_copy(rep_hbm.at[ps.at[ci]], arows, sa)
        hb = pltpu.async_copy(rep_hbm.at[pd.at[ci]], brows, sb)
        hv = pltpu.async_copy(sim_hbm.at[fidx.at[0]], simv.at[0], sv)
        ha.wait()
        hb.wait()
        hv.wait()
        for g in range(8):
            acc = dot_group(g, arows, brows)
            w = jnp.maximum(acc, 0.0)
            sv_ = ps[ci, pl.ds(g * 16, 16)]
            dv_ = pd[ci, pl.ds(g * 16, 16)]
            fs = simv[0, pl.ds(g * 16, 16)]
            pos = fs * THETA + w * (1.0 - THETA)
            m = (sv_ < dv_).astype(jnp.float32)
            pp2 = pp2 + pos * pos * m
            pp1 = pp1 + pos * m
            pc = pc + m
        return pp2, pp1, pc

    def neg_chunk(ci, carry):
        nn2, nc = carry
        ha = pltpu.async_copy(rep_hbm.at[ns_.at[ci]], arows, sa)
        hb = pltpu.async_copy(rep_hbm.at[nd.at[ci]], brows, sb)
        ha.wait()
        hb.wait()
        for g in range(8):
            acc = dot_group(g, arows, brows)
            w = jnp.maximum(acc, 0.0)
            sv_ = ns_[ci, pl.ds(g * 16, 16)]
            dv_ = nd[ci, pl.ds(g * 16, 16)]
            m = (sv_ < dv_).astype(jnp.float32)
            nn2 = nn2 + w * w * m
            nc = nc + m
        return nn2, nc

    pp2, pp1, pc = lax.fori_loop(0, NCH, pos_chunk, (zero16, zero16, zero16))
    nn2, nc = lax.fori_loop(0, NCH, neg_chunk, (zero16, zero16))

    outbuf[0, :] = pp2
    outbuf[1, :] = pp1
    outbuf[2, :] = pc
    outbuf[3, :] = nn2
    outbuf[4, :] = nc
    outbuf[5, :] = zero16
    outbuf[6, :] = zero16
    outbuf[7, :] = zero16
    pltpu.sync_copy(outbuf, out_hbm.at[wid])


def _score(rep, sim_flat, ps_p, pd_p, ns_p, nd_p):
    k = pl.kernel(
        _score_body,
        out_type=jax.ShapeDtypeStruct((NW, 8, 16), jnp.float32),
        mesh=_mesh(),
        scratch_types=[
            pltpu.VMEM((NCH, CH), jnp.int32),
            pltpu.VMEM((NCH, CH), jnp.int32),
            pltpu.VMEM((NCH, CH), jnp.int32),
            pltpu.VMEM((NCH, CH), jnp.int32),
            pltpu.VMEM((CH, F), jnp.float32),
            pltpu.VMEM((CH, F), jnp.float32),
            pltpu.VMEM((1, CH), jnp.int32),
            pltpu.VMEM((1, CH), jnp.float32),
            pltpu.VMEM((8, 16), jnp.float32),
            pltpu.SemaphoreType.DMA,
            pltpu.SemaphoreType.DMA,
            pltpu.SemaphoreType.DMA,
        ],
        compiler_params=pltpu.CompilerParams(needs_layout_passes=False),
    )
    return k(rep, sim_flat, ps_p, pd_p, ns_p, nd_p)


def _tc1_body(x_ref, w_ref, hist_ref, xs_ref, dinv_ref):
    xw = jnp.dot(x_ref[...], w_ref[...], preferred_element_type=jnp.float32)
    deg = jnp.sum(hist_ref[...], axis=0) + 1.0
    dinv = lax.rsqrt(deg)
    xs_ref[...] = xw * dinv[:, None]
    dinv_ref[...] = dinv[:, None]


def _tc1(features, W1, hist):
    return pl.pallas_call(
        _tc1_body,
        grid=(GRID,),
        in_specs=[
            pl.BlockSpec((BLK, F), lambda i: (i, 0)),
            pl.BlockSpec((F, F), lambda i: (0, 0)),
            pl.BlockSpec((NW, BLK), lambda i: (0, i)),
        ],
        out_specs=[
            pl.BlockSpec((BLK, F), lambda i: (i, 0)),
            pl.BlockSpec((BLK, 1), lambda i: (i, 0)),
        ],
        out_shape=[
            jax.ShapeDtypeStruct((NP, F), jnp.float32),
            jax.ShapeDtypeStruct((NP, 1), jnp.float32),
        ],
    )(features, W1, hist)


def _tc2_body(acc_ref, xs1_ref, dinv_ref, b1_ref, w2_ref, xs2_ref):
    dinv = dinv_ref[...]
    ssum = acc_ref[0] + acc_ref[1] + xs1_ref[...]
    h = jnp.maximum(dinv * ssum + b1_ref[...], 0.0)
    xs2_ref[...] = jnp.dot(
        h, w2_ref[...], preferred_element_type=jnp.float32) * dinv


def _tc2(acc1, xs1, dinv, b1, W2):
    return pl.pallas_call(
        _tc2_body,
        grid=(GRID,),
        in_specs=[
            pl.BlockSpec((2, BLK, F), lambda i: (0, i, 0)),
            pl.BlockSpec((BLK, F), lambda i: (i, 0)),
            pl.BlockSpec((BLK, 1), lambda i: (i, 0)),
            pl.BlockSpec((1, F), lambda i: (0, 0)),
            pl.BlockSpec((F, F), lambda i: (0, 0)),
        ],
        out_specs=pl.BlockSpec((BLK, F), lambda i: (i, 0)),
        out_shape=jax.ShapeDtypeStruct((NP, F), jnp.float32),
    )(acc1, xs1, dinv, b1, W2)


def _tc3_body(acc_ref, xs2_ref, dinv_ref, b2_ref, wy_ref, by_ref,
              rep_ref, y_ref):
    dinv = dinv_ref[...]
    h2 = dinv * (acc_ref[0] + acc_ref[1] + xs2_ref[...]) + b2_ref[...]
    n1 = jnp.sqrt(jnp.sum(h2 * h2, axis=1, keepdims=True))
    r1 = h2 / jnp.maximum(n1, 1e-12)
    n2 = jnp.sqrt(jnp.sum(r1 * r1, axis=1, keepdims=True))
    rep = r1 / jnp.maximum(n2, 1e-12)
    rep_ref[...] = rep
    y_ref[...] = jnp.dot(
        rep, wy_ref[...], preferred_element_type=jnp.float32) + by_ref[...]


def _tc3(acc2, xs2, dinv, b2, Wy, by):
    return pl.pallas_call(
        _tc3_body,
        grid=(GRID,),
        in_specs=[
            pl.BlockSpec((2, BLK, F), lambda i: (0, i, 0)),
            pl.BlockSpec((BLK, F), lambda i: (i, 0)),
            pl.BlockSpec((BLK, 1), lambda i: (i, 0)),
            pl.BlockSpec((1, F), lambda i: (0, 0)),
            pl.BlockSpec((F, NCLS), lambda i: (0, 0)),
            pl.BlockSpec((1, NCLS), lambda i: (0, 0)),
        ],
        out_specs=[
            pl.BlockSpec((BLK, F), lambda i: (i, 0)),
            pl.BlockSpec((BLK, NCLS), lambda i: (i, 0)),
        ],
        out_shape=[
            jax.ShapeDtypeStruct((NP, F), jnp.float32),
            jax.ShapeDtypeStruct((NP, NCLS), jnp.float32),
        ],
    )(acc2, xs2, dinv, b2, Wy, by)


def _tc4_body(sc_ref, ls_ref, out_ref):
    arr = sc_ref[...]
    pp2 = jnp.sum(arr[:, 0, :])
    pp1 = jnp.sum(arr[:, 1, :])
    pc = jnp.sum(arr[:, 2, :])
    nn2 = jnp.sum(arr[:, 3, :])
    nc = jnp.sum(arr[:, 4, :])
    ls = ls_ref[0, 0]
    pos_term = pp2 - 2.0 * ls * pp1 + ls * ls * pc
    loss = (nn2 + pos_term) * float(N) / (pc + nc)
    out_ref[...] = jnp.reshape(loss, (1, 1))


def _tc4(score, ls):
    return pl.pallas_call(
        _tc4_body,
        in_specs=[
            pl.BlockSpec(memory_space=pltpu.MemorySpace.VMEM),
            pl.BlockSpec(memory_space=pltpu.MemorySpace.VMEM),
        ],
        out_specs=pl.BlockSpec(memory_space=pltpu.MemorySpace.VMEM),
        out_shape=jax.ShapeDtypeStruct((1, 1), jnp.float32),
    )(score, ls)


def kernel(edge_index, features, sim, label_smoothing,
           W1, b1, W2, b2, Wy, by, neg_edge_index):
    E = edge_index.shape[1]
    pad = EPAD - E
    zpad = jnp.zeros((pad,), jnp.int32)
    jpad = jnp.full((pad,), ACC_ROWS - 1, jnp.int32)

    def shape3(x, p):
        return jnp.concatenate([x, p]).reshape(NW, NCH, CH)

    row_p = shape3(edge_index[0], zpad)
    col_p = shape3(edge_index[1], jpad)
    ps_p = shape3(edge_index[0], zpad)
    pd_p = shape3(edge_index[1], zpad)
    ns_p = shape3(neg_edge_index[0], zpad)
    nd_p = shape3(neg_edge_index[1], zpad)
    sim_flat = sim.reshape(-1)
    feat_p = jnp.concatenate(
        [features, jnp.zeros((NP - N, F), jnp.float32)], axis=0)

    hist = _deg(col_p)
    xs1, dinv = _tc1(feat_p, W1, hist)
    acc1 = _spmm(xs1, row_p, col_p)
    xs2 = _tc2(acc1, xs1, dinv, b1.reshape(1, F), W2)
    acc2 = _spmm(xs2, row_p, col_p)
    rep, y = _tc3(acc2, xs2, dinv, b2.reshape(1, F), Wy, by.reshape(1, NCLS))
    score = _score(rep, sim_flat, ps_p, pd_p, ns_p, nd_p)
    loss = _tc4(score, label_smoothing.reshape(1, 1))
    return (rep[:N], loss.reshape(()), y[:N])


# speedup vs baseline: 3.3366x; 1.3578x over previous
import functools

import jax
import jax.numpy as jnp
from jax import lax
from jax.experimental import pallas as pl
from jax.experimental.pallas import tpu as pltpu
from jax.experimental.pallas import tpu_sc as plsc

N = 10000
F = 128
NCLS = 40
THETA = 0.5

NW = 32
CH = 128
NCH = 80
EPT = NCH * CH
EPAD = NW * EPT
ACC_ROWS = 10240
NP = ACC_ROWS
RPT = ACC_ROWS // 16

BLK = 1280
GRID = NP // BLK


def _mesh():
    return plsc.VectorSubcoreMesh(core_axis_name="c", subcore_axis_name="s")


def _deg_body(col_hbm, out_hbm, cidx, hist):
    c = lax.axis_index("c")
    s = lax.axis_index("s")
    wid = s * 2 + c
    pltpu.sync_copy(col_hbm.at[wid], cidx)
    zero16 = jnp.zeros((16,), jnp.float32)
    ones16 = jnp.ones((16,), jnp.float32)

    def zbody(i, carry):
        hist[pl.ds(i * 16, 16)] = zero16
        return carry

    lax.fori_loop(0, ACC_ROWS // 16, zbody, 0)

    def cbody(ci, carry):
        for g in range(8):
            v = cidx[ci, pl.ds(g * 16, 16)]
            plsc.addupdate_scatter(hist, [v], ones16)
        return carry

    lax.fori_loop(0, NCH, cbody, 0)
    pltpu.sync_copy(hist, out_hbm.at[wid])


def _deg(col_p):
    k = pl.kernel(
        _deg_body,
        out_type=jax.ShapeDtypeStruct((NW, ACC_ROWS), jnp.float32),
        mesh=_mesh(),
        scratch_types=[
            pltpu.VMEM((NCH, CH), jnp.int32),
            pltpu.VMEM((ACC_ROWS,), jnp.float32),
        ],
        compiler_params=pltpu.CompilerParams(needs_layout_passes=False),
    )
    return k(col_p)


IBLK = 8


def _spmm_body(xs_hbm, row_hbm, col_hbm, out_hbm,
               ridx, cidx, buf0, buf1, acc, g0, g1, s0, s1):
    c = lax.axis_index("c")
    s = lax.axis_index("s")
    wid = s * 2 + c

    zero16 = jnp.zeros((16,), jnp.float32)

    def zbody(r, carry):
        for g in range(8):
            buf0[r, pl.ds(g * 16, 16)] = zero16
        return carry

    lax.fori_loop(0, CH, zbody, 0)
    for i in range(RPT // CH):
        pltpu.sync_copy(buf0, acc.at[pl.ds(s * RPT + i * CH, CH)])
    plsc.subcore_barrier()

    bufs = (buf0, buf1)
    gsem = (g0, g1)
    ssem = (s0, s1)
    for blk in range(NCH // IBLK):
        pltpu.sync_copy(row_hbm.at[wid, pl.ds(blk * IBLK, IBLK)], ridx)
        pltpu.sync_copy(col_hbm.at[wid, pl.ds(blk * IBLK, IBLK)], cidx)
        handles = {}
        handles[("g", 0)] = pltpu.async_copy(
            xs_hbm.at[ridx.at[0]], bufs[0], gsem[0])
        for j in range(IBLK):
            b = j & 1
            handles[("g", j)].wait()
            handles[("s", j)] = pltpu.async_copy(
                bufs[b], acc.at[cidx.at[j]], ssem[b], add=True)
            if j + 1 < IBLK:
                if j >= 1:
                    handles[("s", j - 1)].wait()
                handles[("g", j + 1)] = pltpu.async_copy(
                    xs_hbm.at[ridx.at[j + 1]], bufs[1 - b], gsem[1 - b])
        handles[("s", IBLK - 2)].wait()
        handles[("s", IBLK - 1)].wait()
    plsc.subcore_barrier()

    for i in range(RPT // CH):
        rs = s * RPT + i * CH
        pltpu.sync_copy(acc.at[pl.ds(rs, CH)], buf0)
        pltpu.sync_copy(buf0, out_hbm.at[c, pl.ds(rs, CH)])


def _spmm(xs, row_p, col_p):
    k = pl.kernel(
        _spmm_body,
        out_type=jax.ShapeDtypeStruct((2, NP, F), jnp.float32),
        mesh=_mesh(),
        scratch_types=[
            pltpu.VMEM((IBLK, CH), jnp.int32),
            pltpu.VMEM((IBLK, CH), jnp.int32),
            pltpu.VMEM((CH, F), jnp.float32),
            pltpu.VMEM((CH, F), jnp.float32),
            pltpu.VMEM_SHARED((ACC_ROWS, F), jnp.float32),
            pltpu.SemaphoreType.DMA,
            pltpu.SemaphoreType.DMA,
            pltpu.SemaphoreType.DMA,
            pltpu.SemaphoreType.DMA,
        ],
    )
    return k(xs, row_p, col_p)


def _score_body(rep_hbm, sim_hbm, ps_hbm, pd_hbm, ns_hbm, nd_hbm, out_hbm,
                ps, pd, ns_, nd, fidx, a0, b0, a1, b1, simv, outbuf,
                sa0, sb0, sv0, sa1, sb1, sv1):
    c = lax.axis_index("c")
    s = lax.axis_index("s")
    wid = s * 2 + c
    pltpu.sync_copy(ps_hbm.at[wid], ps)
    pltpu.sync_copy(pd_hbm.at[wid], pd)
    pltpu.sync_copy(ns_hbm.at[wid], ns_)
    pltpu.sync_copy(nd_hbm.at[wid], nd)

    zero16 = jnp.zeros((16,), jnp.float32)
    iota16 = lax.iota(jnp.int32, 16)

    def fbody(i, carry):
        for g in range(8):
            sl = pl.ds(g * 16, 16)
            fidx[i, sl] = ps[i, sl] * N + pd[i, sl]
        return carry

    lax.fori_loop(0, NCH, fbody, 0)

    arows = (a0, a1)
    brows = (b0, b1)
    sas = (sa0, sa1)
    sbs = (sb0, sb1)
    svs = (sv0, sv1)

    def issue_pos(ci, k):
        pltpu.async_copy(rep_hbm.at[ps.at[ci]], arows[k], sas[k])
        pltpu.async_copy(rep_hbm.at[pd.at[ci]], brows[k], sbs[k])
        pltpu.async_copy(sim_hbm.at[fidx.at[ci]], simv.at[k], svs[k])

    def issue_neg(ci, k):
        pltpu.async_copy(rep_hbm.at[ns_.at[ci]], arows[k], sas[k])
        pltpu.async_copy(rep_hbm.at[nd.at[ci]], brows[k], sbs[k])

    def wait_rows(k):
        pltpu.make_async_copy(rep_hbm.at[ps.at[0]], arows[k], sas[k]).wait()
        pltpu.make_async_copy(rep_hbm.at[pd.at[0]], brows[k], sbs[k]).wait()

    def wait_sim(k):
        pltpu.make_async_copy(sim_hbm.at[fidx.at[0]], simv.at[k], svs[k]).wait()

    def dots(k, g):
        rows = g * 16 + iota16

        def dbody(q, acc):
            base = q * 16
            for dd in range(16):
                dvec = jnp.full((16,), dd, jnp.int32) + base
                acc = acc + (plsc.load_gather(arows[k], [rows, dvec])
                             * plsc.load_gather(brows[k], [rows, dvec]))
            return acc

        return lax.fori_loop(0, F // 16, dbody, zero16)

    def pos_compute(ci, k, carry):
        def gbody(g, carry):
            pp2, pp1, pc = carry
            w = jnp.maximum(dots(k, g), 0.0)
            sl = pl.ds(g * 16, 16)
            m = (ps[ci, sl] < pd[ci, sl]).astype(jnp.float32)
            pos = simv[k, sl] * THETA + w * (1.0 - THETA)
            return (pp2 + pos * pos * m, pp1 + pos * m, pc + m)

        return lax.fori_loop(0, 8, gbody, carry)

    def neg_compute(ci, k, carry):
        def gbody(g, carry):
            nn2, nc = carry
            w = jnp.maximum(dots(k, g), 0.0)
            sl = pl.ds(g * 16, 16)
            m = (ns_[ci, sl] < nd[ci, sl]).astype(jnp.float32)
            return (nn2 + w * w * m, nc + m)

        return lax.fori_loop(0, 8, gbody, carry)

    issue_pos(0, 0)

    def pos_pair(t, carry):
        issue_pos(2 * t + 1, 1)
        wait_rows(0)
        wait_sim(0)
        carry = pos_compute(2 * t, 0, carry)

        @pl.when(2 * t + 2 < NCH)
        def _():
            issue_pos(2 * t + 2, 0)

        wait_rows(1)
        wait_sim(1)
        return pos_compute(2 * t + 1, 1, carry)

    pp2, pp1, pc = lax.fori_loop(
        0, NCH // 2, pos_pair, (zero16, zero16, zero16))

    issue_neg(0, 0)

    def neg_pair(t, carry):
        issue_neg(2 * t + 1, 1)
        wait_rows(0)
        carry = neg_compute(2 * t, 0, carry)

        @pl.when(2 * t + 2 < NCH)
        def _():
            issue_neg(2 * t + 2, 0)

        wait_rows(1)
        return neg_compute(2 * t + 1, 1, carry)

    nn2, nc = lax.fori_loop(0, NCH // 2, neg_pair, (zero16, zero16))

    outbuf[0, :] = pp2
    outbuf[1, :] = pp1
    outbuf[2, :] = pc
    outbuf[3, :] = nn2
    outbuf[4, :] = nc
    outbuf[5, :] = zero16
    outbuf[6, :] = zero16
    outbuf[7, :] = zero16
    pltpu.sync_copy(outbuf, out_hbm.at[wid])


def _score(rep, sim_flat, ps_p, pd_p, ns_p, nd_p):
    k = pl.kernel(
        _score_body,
        out_type=jax.ShapeDtypeStruct((NW, 8, 16), jnp.float32),
        mesh=_mesh(),
        scratch_types=[
            pltpu.VMEM((NCH, CH), jnp.int32),
            pltpu.VMEM((NCH, CH), jnp.int32),
            pltpu.VMEM((NCH, CH), jnp.int32),
            pltpu.VMEM((NCH, CH), jnp.int32),
            pltpu.VMEM((NCH, CH), jnp.int32),
            pltpu.VMEM((CH, F), jnp.float32),
            pltpu.VMEM((CH, F), jnp.float32),
            pltpu.VMEM((CH, F), jnp.float32),
            pltpu.VMEM((CH, F), jnp.float32),
            pltpu.VMEM((2, CH), jnp.float32),
            pltpu.VMEM((8, 16), jnp.float32),
            pltpu.SemaphoreType.DMA,
            pltpu.SemaphoreType.DMA,
            pltpu.SemaphoreType.DMA,
            pltpu.SemaphoreType.DMA,
            pltpu.SemaphoreType.DMA,
            pltpu.SemaphoreType.DMA,
        ],
        compiler_params=pltpu.CompilerParams(needs_layout_passes=False),
    )
    return k(rep, sim_flat, ps_p, pd_p, ns_p, nd_p)


def _tc1_body(x_ref, w_ref, hist_ref, xs_ref, dinv_ref):
    xw = jnp.dot(x_ref[...], w_ref[...], preferred_element_type=jnp.float32)
    deg = jnp.sum(hist_ref[...], axis=0) + 1.0
    dinv = lax.rsqrt(deg)
    xs_ref[...] = xw * dinv[:, None]
    dinv_ref[...] = dinv[:, None]


def _tc1(features, W1, hist):
    return pl.pallas_call(
        _tc1_body,
        grid=(GRID,),
        in_specs=[
            pl.BlockSpec((BLK, F), lambda i: (i, 0)),
            pl.BlockSpec((F, F), lambda i: (0, 0)),
            pl.BlockSpec((NW, BLK), lambda i: (0, i)),
        ],
        out_specs=[
            pl.BlockSpec((BLK, F), lambda i: (i, 0)),
            pl.BlockSpec((BLK, 1), lambda i: (i, 0)),
        ],
        out_shape=[
            jax.ShapeDtypeStruct((NP, F), jnp.float32),
            jax.ShapeDtypeStruct((NP, 1), jnp.float32),
        ],
    )(features, W1, hist)


def _tc2_body(acc_ref, xs1_ref, dinv_ref, b1_ref, w2_ref, xs2_ref):
    dinv = dinv_ref[...]
    ssum = acc_ref[0] + acc_ref[1] + xs1_ref[...]
    h = jnp.maximum(dinv * ssum + b1_ref[...], 0.0)
    xs2_ref[...] = jnp.dot(
        h, w2_ref[...], preferred_element_type=jnp.float32) * dinv


def _tc2(acc1, xs1, dinv, b1, W2):
    return pl.pallas_call(
        _tc2_body,
        grid=(GRID,),
        in_specs=[
            pl.BlockSpec((2, BLK, F), lambda i: (0, i, 0)),
            pl.BlockSpec((BLK, F), lambda i: (i, 0)),
            pl.BlockSpec((BLK, 1), lambda i: (i, 0)),
            pl.BlockSpec((1, F), lambda i: (0, 0)),
            pl.BlockSpec((F, F), lambda i: (0, 0)),
        ],
        out_specs=pl.BlockSpec((BLK, F), lambda i: (i, 0)),
        out_shape=jax.ShapeDtypeStruct((NP, F), jnp.float32),
    )(acc1, xs1, dinv, b1, W2)


def _tc3_body(acc_ref, xs2_ref, dinv_ref, b2_ref, wy_ref, by_ref,
              rep_ref, y_ref):
    dinv = dinv_ref[...]
    h2 = dinv * (acc_ref[0] + acc_ref[1] + xs2_ref[...]) + b2_ref[...]
    n1 = jnp.sqrt(jnp.sum(h2 * h2, axis=1, keepdims=True))
    r1 = h2 / jnp.maximum(n1, 1e-12)
    n2 = jnp.sqrt(jnp.sum(r1 * r1, axis=1, keepdims=True))
    rep = r1 / jnp.maximum(n2, 1e-12)
    rep_ref[...] = rep
    y_ref[...] = jnp.dot(
        rep, wy_ref[...], preferred_element_type=jnp.float32) + by_ref[...]


def _tc3(acc2, xs2, dinv, b2, Wy, by):
    return pl.pallas_call(
        _tc3_body,
        grid=(GRID,),
        in_specs=[
            pl.BlockSpec((2, BLK, F), lambda i: (0, i, 0)),
            pl.BlockSpec((BLK, F), lambda i: (i, 0)),
            pl.BlockSpec((BLK, 1), lambda i: (i, 0)),
            pl.BlockSpec((1, F), lambda i: (0, 0)),
            pl.BlockSpec((F, NCLS), lambda i: (0, 0)),
            pl.BlockSpec((1, NCLS), lambda i: (0, 0)),
        ],
        out_specs=[
            pl.BlockSpec((BLK, F), lambda i: (i, 0)),
            pl.BlockSpec((BLK, NCLS), lambda i: (i, 0)),
        ],
        out_shape=[
            jax.ShapeDtypeStruct((NP, F), jnp.float32),
            jax.ShapeDtypeStruct((NP, NCLS), jnp.float32),
        ],
    )(acc2, xs2, dinv, b2, Wy, by)


def _tc4_body(sc_ref, ls_ref, out_ref):
    arr = sc_ref[...]
    pp2 = jnp.sum(arr[:, 0, :])
    pp1 = jnp.sum(arr[:, 1, :])
    pc = jnp.sum(arr[:, 2, :])
    nn2 = jnp.sum(arr[:, 3, :])
    nc = jnp.sum(arr[:, 4, :])
    ls = ls_ref[0, 0]
    pos_term = pp2 - 2.0 * ls * pp1 + ls * ls * pc
    loss = (nn2 + pos_term) * float(N) / (pc + nc)
    out_ref[...] = jnp.reshape(loss, (1, 1))


def _tc4(score, ls):
    return pl.pallas_call(
        _tc4_body,
        in_specs=[
            pl.BlockSpec(memory_space=pltpu.MemorySpace.VMEM),
            pl.BlockSpec(memory_space=pltpu.MemorySpace.VMEM),
        ],
        out_specs=pl.BlockSpec(memory_space=pltpu.MemorySpace.VMEM),
        out_shape=jax.ShapeDtypeStruct((1, 1), jnp.float32),
    )(score, ls)


def kernel(edge_index, features, sim, label_smoothing,
           W1, b1, W2, b2, Wy, by, neg_edge_index):
    E = edge_index.shape[1]
    pad = EPAD - E
    zpad = jnp.zeros((pad,), jnp.int32)
    jpad = jnp.full((pad,), ACC_ROWS - 1, jnp.int32)

    def shape3(x, p):
        return jnp.concatenate([x, p]).reshape(NW, NCH, CH)

    row_p = shape3(edge_index[0], zpad)
    col_p = shape3(edge_index[1], jpad)
    ps_p = shape3(edge_index[0], zpad)
    pd_p = shape3(edge_index[1], zpad)
    ns_p = shape3(neg_edge_index[0], zpad)
    nd_p = shape3(neg_edge_index[1], zpad)
    sim_flat = sim.reshape(-1)
    feat_p = jnp.concatenate(
        [features, jnp.zeros((NP - N, F), jnp.float32)], axis=0)

    hist = _deg(col_p)
    xs1, dinv = _tc1(feat_p, W1, hist)
    acc1 = _spmm(xs1, row_p, col_p)
    xs2 = _tc2(acc1, xs1, dinv, b1.reshape(1, F), W2)
    acc2 = _spmm(xs2, row_p, col_p)
    rep, y = _tc3(acc2, xs2, dinv, b2.reshape(1, F), Wy, by.reshape(1, NCLS))
    score = _score(rep, sim_flat, ps_p, pd_p, ns_p, nd_p)
    loss = _tc4(score, label_smoothing.reshape(1, 1))
    return (rep[:N], loss.reshape(()), y[:N])


# speedup vs baseline: 4.5495x; 1.3635x over previous
import functools

import jax
import jax.numpy as jnp
from jax import lax
from jax.experimental import pallas as pl
from jax.experimental.pallas import tpu as pltpu
from jax.experimental.pallas import tpu_sc as plsc

N = 10000
F = 128
NCLS = 40
THETA = 0.5

NW = 32
CH = 128
NCH = 80
EPT = NCH * CH
EPAD = NW * EPT
ACC_ROWS = 10240
NP = ACC_ROWS
RPT = ACC_ROWS // 16

BLK = 1280
GRID = NP // BLK


def _mesh():
    return plsc.VectorSubcoreMesh(core_axis_name="c", subcore_axis_name="s")


def _deg_body(col_hbm, out_hbm, cidx, hist):
    c = lax.axis_index("c")
    s = lax.axis_index("s")
    wid = s * 2 + c
    pltpu.sync_copy(col_hbm.at[wid], cidx)
    zero16 = jnp.zeros((16,), jnp.float32)
    ones16 = jnp.ones((16,), jnp.float32)

    def zbody(i, carry):
        hist[pl.ds(i * 16, 16)] = zero16
        return carry

    lax.fori_loop(0, ACC_ROWS // 16, zbody, 0)

    def cbody(ci, carry):
        for g in range(8):
            v = cidx[ci, pl.ds(g * 16, 16)]
            plsc.addupdate_scatter(hist, [v], ones16)
        return carry

    lax.fori_loop(0, NCH, cbody, 0)
    pltpu.sync_copy(hist, out_hbm.at[wid])


def _deg(col_p):
    k = pl.kernel(
        _deg_body,
        out_type=jax.ShapeDtypeStruct((NW, ACC_ROWS), jnp.float32),
        mesh=_mesh(),
        scratch_types=[
            pltpu.VMEM((NCH, CH), jnp.int32),
            pltpu.VMEM((ACC_ROWS,), jnp.float32),
        ],
        compiler_params=pltpu.CompilerParams(needs_layout_passes=False),
    )
    return k(col_p)


IBLK = 8


def _spmm_body(xs_hbm, row_hbm, col_hbm, out_hbm,
               ridx, cidx, buf0, buf1, acc, g0, g1, s0, s1):
    c = lax.axis_index("c")
    s = lax.axis_index("s")
    wid = s * 2 + c

    zero16 = jnp.zeros((16,), jnp.float32)

    def zbody(r, carry):
        for g in range(8):
            buf0[r, pl.ds(g * 16, 16)] = zero16
        return carry

    lax.fori_loop(0, CH, zbody, 0)
    for i in range(RPT // CH):
        pltpu.sync_copy(buf0, acc.at[pl.ds(s * RPT + i * CH, CH)])
    plsc.subcore_barrier()

    bufs = (buf0, buf1)
    gsem = (g0, g1)
    ssem = (s0, s1)
    for blk in range(NCH // IBLK):
        pltpu.sync_copy(row_hbm.at[wid, pl.ds(blk * IBLK, IBLK)], ridx)
        pltpu.sync_copy(col_hbm.at[wid, pl.ds(blk * IBLK, IBLK)], cidx)
        handles = {}
        handles[("g", 0)] = pltpu.async_copy(
            xs_hbm.at[ridx.at[0]], bufs[0], gsem[0])
        for j in range(IBLK):
            b = j & 1
            handles[("g", j)].wait()
            handles[("s", j)] = pltpu.async_copy(
                bufs[b], acc.at[cidx.at[j]], ssem[b], add=True)
            if j + 1 < IBLK:
                if j >= 1:
                    handles[("s", j - 1)].wait()
                handles[("g", j + 1)] = pltpu.async_copy(
                    xs_hbm.at[ridx.at[j + 1]], bufs[1 - b], gsem[1 - b])
        handles[("s", IBLK - 2)].wait()
        handles[("s", IBLK - 1)].wait()
    plsc.subcore_barrier()

    for i in range(RPT // CH):
        rs = s * RPT + i * CH
        pltpu.sync_copy(acc.at[pl.ds(rs, CH)], buf0)
        pltpu.sync_copy(buf0, out_hbm.at[c, pl.ds(rs, CH)])


def _spmm(xs, row_p, col_p):
    k = pl.kernel(
        _spmm_body,
        out_type=jax.ShapeDtypeStruct((2, NP, F), jnp.float32),
        mesh=_mesh(),
        scratch_types=[
            pltpu.VMEM((IBLK, CH), jnp.int32),
            pltpu.VMEM((IBLK, CH), jnp.int32),
            pltpu.VMEM((CH, F), jnp.float32),
            pltpu.VMEM((CH, F), jnp.float32),
            pltpu.VMEM_SHARED((ACC_ROWS, F), jnp.float32),
            pltpu.SemaphoreType.DMA,
            pltpu.SemaphoreType.DMA,
            pltpu.SemaphoreType.DMA,
            pltpu.SemaphoreType.DMA,
        ],
    )
    return k(xs, row_p, col_p)


def _score_body(rep_hbm, sim_hbm, ps_hbm, pd_hbm, ns_hbm, nd_hbm, out_hbm,
                ps, pd, ns_, nd, fidx, a0, b0, a1, b1, simv, tmp, outbuf,
                sa0, sb0, sv0, sa1, sb1, sv1):
    c = lax.axis_index("c")
    s = lax.axis_index("s")
    wid = s * 2 + c
    pltpu.sync_copy(ps_hbm.at[wid], ps)
    pltpu.sync_copy(pd_hbm.at[wid], pd)
    pltpu.sync_copy(ns_hbm.at[wid], ns_)
    pltpu.sync_copy(nd_hbm.at[wid], nd)

    zero16 = jnp.zeros((16,), jnp.float32)
    iota16 = lax.iota(jnp.int32, 16)

    def fbody(i, carry):
        for g in range(8):
            sl = pl.ds(g * 16, 16)
            fidx[i, sl] = ps[i, sl] * N + pd[i, sl]
        return carry

    lax.fori_loop(0, NCH, fbody, 0)

    arows = (a0, a1)
    brows = (b0, b1)
    sas = (sa0, sa1)
    sbs = (sb0, sb1)
    svs = (sv0, sv1)

    def issue_pos(ci, k):
        pltpu.async_copy(rep_hbm.at[ps.at[ci]], arows[k], sas[k])
        pltpu.async_copy(rep_hbm.at[pd.at[ci]], brows[k], sbs[k])
        pltpu.async_copy(sim_hbm.at[fidx.at[ci]], simv.at[k], svs[k])

    def issue_neg(ci, k):
        pltpu.async_copy(rep_hbm.at[ns_.at[ci]], arows[k], sas[k])
        pltpu.async_copy(rep_hbm.at[nd.at[ci]], brows[k], sbs[k])

    def wait_rows(k):
        pltpu.make_async_copy(rep_hbm.at[ps.at[0]], arows[k], sas[k]).wait()
        pltpu.make_async_copy(rep_hbm.at[pd.at[0]], brows[k], sbs[k]).wait()

    def wait_sim(k):
        pltpu.make_async_copy(sim_hbm.at[fidx.at[0]], simv.at[k], svs[k]).wait()

    def dots(k, g):
        base = g * 16
        for e in range(16):
            row = base + e
            pa = (arows[k][row, pl.ds(0, 16)]
                  * brows[k][row, pl.ds(0, 16)])
            for j in range(1, 8):
                pa = pa + (arows[k][row, pl.ds(j * 16, 16)]
                           * brows[k][row, pl.ds(j * 16, 16)])
            tmp[e, pl.ds(0, 16)] = pa
        w = zero16
        for cc in range(16):
            w = w + plsc.load_gather(tmp, [iota16, jnp.full((16,), cc, jnp.int32)])
        return w

    def pos_compute(ci, k, carry):
        def gbody(g, carry):
            pp2, pp1, pc = carry
            w = jnp.maximum(dots(k, g), 0.0)
            sl = pl.ds(g * 16, 16)
            m = (ps[ci, sl] < pd[ci, sl]).astype(jnp.float32)
            pos = simv[k, sl] * THETA + w * (1.0 - THETA)
            return (pp2 + pos * pos * m, pp1 + pos * m, pc + m)

        return lax.fori_loop(0, 8, gbody, carry)

    def neg_compute(ci, k, carry):
        def gbody(g, carry):
            nn2, nc = carry
            w = jnp.maximum(dots(k, g), 0.0)
            sl = pl.ds(g * 16, 16)
            m = (ns_[ci, sl] < nd[ci, sl]).astype(jnp.float32)
            return (nn2 + w * w * m, nc + m)

        return lax.fori_loop(0, 8, gbody, carry)

    issue_pos(0, 0)

    def pos_pair(t, carry):
        issue_pos(2 * t + 1, 1)
        wait_rows(0)
        wait_sim(0)
        carry = pos_compute(2 * t, 0, carry)

        @pl.when(2 * t + 2 < NCH)
        def _():
            issue_pos(2 * t + 2, 0)

        wait_rows(1)
        wait_sim(1)
        return pos_compute(2 * t + 1, 1, carry)

    pp2, pp1, pc = lax.fori_loop(
        0, NCH // 2, pos_pair, (zero16, zero16, zero16))

    issue_neg(0, 0)

    def neg_pair(t, carry):
        issue_neg(2 * t + 1, 1)
        wait_rows(0)
        carry = neg_compute(2 * t, 0, carry)

        @pl.when(2 * t + 2 < NCH)
        def _():
            issue_neg(2 * t + 2, 0)

        wait_rows(1)
        return neg_compute(2 * t + 1, 1, carry)

    nn2, nc = lax.fori_loop(0, NCH // 2, neg_pair, (zero16, zero16))

    outbuf[0, :] = pp2
    outbuf[1, :] = pp1
    outbuf[2, :] = pc
    outbuf[3, :] = nn2
    outbuf[4, :] = nc
    outbuf[5, :] = zero16
    outbuf[6, :] = zero16
    outbuf[7, :] = zero16
    pltpu.sync_copy(outbuf, out_hbm.at[wid])


def _score(rep, sim_flat, ps_p, pd_p, ns_p, nd_p):
    k = pl.kernel(
        _score_body,
        out_type=jax.ShapeDtypeStruct((NW, 8, 16), jnp.float32),
        mesh=_mesh(),
        scratch_types=[
            pltpu.VMEM((NCH, CH), jnp.int32),
            pltpu.VMEM((NCH, CH), jnp.int32),
            pltpu.VMEM((NCH, CH), jnp.int32),
            pltpu.VMEM((NCH, CH), jnp.int32),
            pltpu.VMEM((NCH, CH), jnp.int32),
            pltpu.VMEM((CH, F), jnp.float32),
            pltpu.VMEM((CH, F), jnp.float32),
            pltpu.VMEM((CH, F), jnp.float32),
            pltpu.VMEM((CH, F), jnp.float32),
            pltpu.VMEM((2, CH), jnp.float32),
            pltpu.VMEM((16, 17), jnp.float32),
            pltpu.VMEM((8, 16), jnp.float32),
            pltpu.SemaphoreType.DMA,
            pltpu.SemaphoreType.DMA,
            pltpu.SemaphoreType.DMA,
            pltpu.SemaphoreType.DMA,
            pltpu.SemaphoreType.DMA,
            pltpu.SemaphoreType.DMA,
        ],
        compiler_params=pltpu.CompilerParams(needs_layout_passes=False),
    )
    return k(rep, sim_flat, ps_p, pd_p, ns_p, nd_p)


def _tc1_body(x_ref, w_ref, hist_ref, xs_ref, dinv_ref):
    xw = jnp.dot(x_ref[...], w_ref[...], preferred_element_type=jnp.float32)
    deg = jnp.sum(hist_ref[...], axis=0) + 1.0
    dinv = lax.rsqrt(deg)
    xs_ref[...] = xw * dinv[:, None]
    dinv_ref[...] = dinv[:, None]


def _tc1(features, W1, hist):
    return pl.pallas_call(
        _tc1_body,
        grid=(GRID,),
        in_specs=[
            pl.BlockSpec((BLK, F), lambda i: (i, 0)),
            pl.BlockSpec((F, F), lambda i: (0, 0)),
            pl.BlockSpec((NW, BLK), lambda i: (0, i)),
        ],
        out_specs=[
            pl.BlockSpec((BLK, F), lambda i: (i, 0)),
            pl.BlockSpec((BLK, 1), lambda i: (i, 0)),
        ],
        out_shape=[
            jax.ShapeDtypeStruct((NP, F), jnp.float32),
            jax.ShapeDtypeStruct((NP, 1), jnp.float32),
        ],
    )(features, W1, hist)


def _tc2_body(acc_ref, xs1_ref, dinv_ref, b1_ref, w2_ref, xs2_ref):
    dinv = dinv_ref[...]
    ssum = acc_ref[0] + acc_ref[1] + xs1_ref[...]
    h = jnp.maximum(dinv * ssum + b1_ref[...], 0.0)
    xs2_ref[...] = jnp.dot(
        h, w2_ref[...], preferred_element_type=jnp.float32) * dinv


def _tc2(acc1, xs1, dinv, b1, W2):
    return pl.pallas_call(
        _tc2_body,
        grid=(GRID,),
        in_specs=[
            pl.BlockSpec((2, BLK, F), lambda i: (0, i, 0)),
            pl.BlockSpec((BLK, F), lambda i: (i, 0)),
            pl.BlockSpec((BLK, 1), lambda i: (i, 0)),
            pl.BlockSpec((1, F), lambda i: (0, 0)),
            pl.BlockSpec((F, F), lambda i: (0, 0)),
        ],
        out_specs=pl.BlockSpec((BLK, F), lambda i: (i, 0)),
        out_shape=jax.ShapeDtypeStruct((NP, F), jnp.float32),
    )(acc1, xs1, dinv, b1, W2)


def _tc3_body(acc_ref, xs2_ref, dinv_ref, b2_ref, wy_ref, by_ref,
              rep_ref, y_ref):
    dinv = dinv_ref[...]
    h2 = dinv * (acc_ref[0] + acc_ref[1] + xs2_ref[...]) + b2_ref[...]
    n1 = jnp.sqrt(jnp.sum(h2 * h2, axis=1, keepdims=True))
    r1 = h2 / jnp.maximum(n1, 1e-12)
    n2 = jnp.sqrt(jnp.sum(r1 * r1, axis=1, keepdims=True))
    rep = r1 / jnp.maximum(n2, 1e-12)
    rep_ref[...] = rep
    y_ref[...] = jnp.dot(
        rep, wy_ref[...], preferred_element_type=jnp.float32) + by_ref[...]


def _tc3(acc2, xs2, dinv, b2, Wy, by):
    return pl.pallas_call(
        _tc3_body,
        grid=(GRID,),
        in_specs=[
            pl.BlockSpec((2, BLK, F), lambda i: (0, i, 0)),
            pl.BlockSpec((BLK, F), lambda i: (i, 0)),
            pl.BlockSpec((BLK, 1), lambda i: (i, 0)),
            pl.BlockSpec((1, F), lambda i: (0, 0)),
            pl.BlockSpec((F, NCLS), lambda i: (0, 0)),
            pl.BlockSpec((1, NCLS), lambda i: (0, 0)),
        ],
        out_specs=[
            pl.BlockSpec((BLK, F), lambda i: (i, 0)),
            pl.BlockSpec((BLK, NCLS), lambda i: (i, 0)),
        ],
        out_shape=[
            jax.ShapeDtypeStruct((NP, F), jnp.float32),
            jax.ShapeDtypeStruct((NP, NCLS), jnp.float32),
        ],
    )(acc2, xs2, dinv, b2, Wy, by)


def _tc4_body(sc_ref, ls_ref, out_ref):
    arr = sc_ref[...]
    pp2 = jnp.sum(arr[:, 0, :])
    pp1 = jnp.sum(arr[:, 1, :])
    pc = jnp.sum(arr[:, 2, :])
    nn2 = jnp.sum(arr[:, 3, :])
    nc = jnp.sum(arr[:, 4, :])
    ls = ls_ref[0, 0]
    pos_term = pp2 - 2.0 * ls * pp1 + ls * ls * pc
    loss = (nn2 + pos_term) * float(N) / (pc + nc)
    out_ref[...] = jnp.reshape(loss, (1, 1))


def _tc4(score, ls):
    return pl.pallas_call(
        _tc4_body,
        in_specs=[
            pl.BlockSpec(memory_space=pltpu.MemorySpace.VMEM),
            pl.BlockSpec(memory_space=pltpu.MemorySpace.VMEM),
        ],
        out_specs=pl.BlockSpec(memory_space=pltpu.MemorySpace.VMEM),
        out_shape=jax.ShapeDtypeStruct((1, 1), jnp.float32),
    )(score, ls)


def kernel(edge_index, features, sim, label_smoothing,
           W1, b1, W2, b2, Wy, by, neg_edge_index):
    E = edge_index.shape[1]
    pad = EPAD - E
    zpad = jnp.zeros((pad,), jnp.int32)
    jpad = jnp.full((pad,), ACC_ROWS - 1, jnp.int32)

    def shape3(x, p):
        return jnp.concatenate([x, p]).reshape(NW, NCH, CH)

    row_p = shape3(edge_index[0], zpad)
    col_p = shape3(edge_index[1], jpad)
    ps_p = shape3(edge_index[0], zpad)
    pd_p = shape3(edge_index[1], zpad)
    ns_p = shape3(neg_edge_index[0], zpad)
    nd_p = shape3(neg_edge_index[1], zpad)
    sim_flat = sim.reshape(-1)
    feat_p = jnp.concatenate(
        [features, jnp.zeros((NP - N, F), jnp.float32)], axis=0)

    hist = _deg(col_p)
    xs1, dinv = _tc1(feat_p, W1, hist)
    acc1 = _spmm(xs1, row_p, col_p)
    xs2 = _tc2(acc1, xs1, dinv, b1.reshape(1, F), W2)
    acc2 = _spmm(xs2, row_p, col_p)
    rep, y = _tc3(acc2, xs2, dinv, b2.reshape(1, F), Wy, by.reshape(1, NCLS))
    score = _score(rep, sim_flat, ps_p, pd_p, ns_p, nd_p)
    loss = _tc4(score, label_smoothing.reshape(1, 1))
    return (rep[:N], loss.reshape(()), y[:N])
